# baseline (speedup 1.0000x reference)
import functools

import numpy as np
import jax
import jax.numpy as jnp
from jax import lax
from jax.experimental import pallas as pl
from jax.experimental.pallas import tpu as pltpu
from jax.experimental.pallas import tpu_sc as plsc

_B, _H, _W, _C = 4, 224, 224, 96
_OUT_SHAPE = (224, 224)
_CENTER = (112.0, 112.0)
_MAX_RADIUS = 112.0

_NW = 32
_NROWS = _B * _H * _W
_PER_W = _NROWS // _NW
_CHUNK = 112
_NCHUNK = _PER_W // _CHUNK

_tables_cache = None


def _build_tables():
    global _tables_cache
    if _tables_cache is not None:
        return _tables_cache
    radius = np.linspace(0.0, _MAX_RADIUS, _OUT_SHAPE[0], dtype=np.float32).astype(np.float64)
    theta = np.linspace(0.0, 2.0 * np.pi, _OUT_SHAPE[1], endpoint=False,
                        dtype=np.float32).astype(np.float64)
    c1 = _CENTER[0] + radius[:, None] * np.cos(theta)[None, :]
    c2 = _CENTER[1] + radius[:, None] * np.sin(theta)[None, :]
    x0f = np.floor(c1); fx = c1 - x0f
    y0f = np.floor(c2); fy = c2 - y0f
    x0 = np.clip(x0f, 0, _H - 1).astype(np.int64)
    x1 = np.clip(x0f + 1, 0, _H - 1).astype(np.int64)
    y0 = np.clip(y0f, 0, _W - 1).astype(np.int64)
    y1 = np.clip(y0f + 1, 0, _W - 1).astype(np.int64)
    i = np.stack([(x0 * _W + y0).ravel(), (x0 * _W + y1).ravel(),
                  (x1 * _W + y0).ravel(), (x1 * _W + y1).ravel()])
    w = np.stack([((1 - fx) * (1 - fy)).ravel(), ((1 - fx) * fy).ravel(),
                  (fx * (1 - fy)).ravel(), (fx * fy).ravel()])
    boff = (np.arange(_B) * (_H * _W))[None, :, None]
    idx = (i[:, None, :] + boff).reshape(4 * _NROWS).astype(np.int32)
    wts = np.zeros((_NROWS, 8), dtype=np.float32)
    wts[:, :4] = np.broadcast_to(w.T[None], (_B, _H * _W, 4)).reshape(_NROWS, 4)
    _tables_cache = (idx, wts.reshape(-1))
    return _tables_cache


_CP = 128


def _compact_body(rows128_hbm, rows_hbm):
    wid = lax.axis_index("s") * 2 + lax.axis_index("c")
    n = _PER_W // 4
    for q in range(4):
        r0 = wid * _PER_W + q * n
        pltpu.sync_copy(rows128_hbm.at[pl.ds(r0, n), pl.ds(0, _C)],
                        rows_hbm.at[pl.ds(r0, n)])


@jax.jit
def _compact_sc(rows128):
    mesh = plsc.VectorSubcoreMesh(core_axis_name="c", subcore_axis_name="s")
    f = functools.partial(
        pl.kernel,
        mesh=mesh,
        out_type=jax.ShapeDtypeStruct((_NROWS, _C), jnp.float32),
        compiler_params=pltpu.CompilerParams(use_tc_tiling_on_sc=False),
    )(_compact_body)
    return f(rows128)


def _sc_body(rows_hbm, idx_hbm, wts_hbm, out_hbm,
             idx_v0, idx_v1, w_v0, w_v1, tap_v0, tap_v1, out_v0, out_v1,
             sem0, sem1):
    wid = lax.axis_index("s") * 2 + lax.axis_index("c")
    bufs = ((idx_v0, w_v0, tap_v0, out_v0, sem0),
            (idx_v1, w_v1, tap_v1, out_v1, sem1))

    def start(j, b):
        idx_v, w_v, tap_v, _, sem = bufs[b]
        base = wid * _PER_W + j * _CHUNK
        for t in range(4):
            pltpu.sync_copy(idx_hbm.at[pl.ds(t * _NROWS + base, _CHUNK)],
                            idx_v.at[pl.ds(t * _CHUNK, _CHUNK)])
        pltpu.sync_copy(wts_hbm.at[pl.ds(base * 8, _CHUNK * 8)],
                        w_v.at[pl.ds(0, _CHUNK * 8)])
        for t in range(4):
            pltpu.async_copy(rows_hbm.at[idx_v.at[pl.ds(t * _CHUNK, _CHUNK)]],
                             tap_v.at[t], sem)

    def finish(j, b):
        idx_v, w_v, tap_v, out_v, sem = bufs[b]
        base = wid * _PER_W + j * _CHUNK
        for t in range(4):
            pltpu.make_async_copy(
                rows_hbm.at[idx_v.at[pl.ds(t * _CHUNK, _CHUNK)]],
                tap_v.at[t], sem).wait()

        def row(k, c2):
            wvec = w_v[pl.ds(k * 8, 16)]
            w0 = wvec[0]; w1 = wvec[1]; w2 = wvec[2]; w3 = wvec[3]
            for c in range(_C // 16):
                s = pl.ds(c * 16, 16)
                out_v[k, s] = (tap_v[0, k, s] * w0 + tap_v[1, k, s] * w1
                               + tap_v[2, k, s] * w2 + tap_v[3, k, s] * w3)
            return c2

        lax.fori_loop(0, _CHUNK, row, 0)
        pltpu.sync_copy(out_v, out_hbm.at[pl.ds(base, _CHUNK)])

    start(0, 0)

    def pair(j2, carry):
        e = j2 * 2
        start(e + 1, 1)
        finish(e, 0)

        @pl.when(j2 < _NCHUNK // 2 - 1)
        def _():
            start(e + 2, 0)

        finish(e + 1, 1)
        return carry

    lax.fori_loop(0, _NCHUNK // 2, pair, 0)


@jax.jit
def _polar_sc(rows, idx, wts):
    mesh = plsc.VectorSubcoreMesh(core_axis_name="c", subcore_axis_name="s")
    f = functools.partial(
        pl.kernel,
        mesh=mesh,
        out_type=jax.ShapeDtypeStruct((_NROWS, _CP), jnp.float32),
        scratch_types=[
            pltpu.VMEM((4 * _CHUNK,), jnp.int32),
            pltpu.VMEM((4 * _CHUNK,), jnp.int32),
            pltpu.VMEM((_CHUNK * 8 + 16,), jnp.float32),
            pltpu.VMEM((_CHUNK * 8 + 16,), jnp.float32),
            pltpu.VMEM((4, _CHUNK, _C), jnp.float32),
            pltpu.VMEM((4, _CHUNK, _C), jnp.float32),
            pltpu.VMEM((_CHUNK, _CP), jnp.float32),
            pltpu.VMEM((_CHUNK, _CP), jnp.float32),
            pltpu.SemaphoreType.DMA,
            pltpu.SemaphoreType.DMA,
        ],
        compiler_params=pltpu.CompilerParams(use_tc_tiling_on_sc=False),
    )(_sc_body)
    return f(rows, idx, wts)


def kernel(inputs):
    assert inputs.shape == (_B, _H, _W, _C)
    rows128 = jnp.pad(inputs, ((0, 0), (0, 0), (0, 0), (0, _CP - _C))).reshape(_NROWS, _CP)
    rows = _compact_sc(rows128)
    idx_np, wts_np = _build_tables()
    out = _polar_sc(rows, jnp.asarray(idx_np), jnp.asarray(wts_np))
    return out[:, :_C].reshape(_B, _H, _W, _C)


# speedup vs baseline: 3.4854x; 3.4854x over previous
import functools

import numpy as np
import jax
import jax.numpy as jnp
from jax import lax
from jax.experimental import pallas as pl
from jax.experimental.pallas import tpu as pltpu
from jax.experimental.pallas import tpu_sc as plsc

_B, _H, _W, _C = 4, 224, 224, 96
_OUT_SHAPE = (224, 224)
_CENTER = (112.0, 112.0)
_MAX_RADIUS = 112.0

_NW = 32
_NROWS = _B * _H * _W
_PER_W = _NROWS // _NW
_CHUNK = 64
_NCHUNK = _PER_W // _CHUNK

_tables_cache = None


def _build_tables():
    global _tables_cache
    if _tables_cache is not None:
        return _tables_cache
    radius = np.linspace(0.0, _MAX_RADIUS, _OUT_SHAPE[0], dtype=np.float32).astype(np.float64)
    theta = np.linspace(0.0, 2.0 * np.pi, _OUT_SHAPE[1], endpoint=False,
                        dtype=np.float32).astype(np.float64)
    c1 = _CENTER[0] + radius[:, None] * np.cos(theta)[None, :]
    c2 = _CENTER[1] + radius[:, None] * np.sin(theta)[None, :]
    x0f = np.floor(c1); fx = c1 - x0f
    y0f = np.floor(c2); fy = c2 - y0f
    x0 = np.clip(x0f, 0, _H - 1).astype(np.int64)
    x1 = np.clip(x0f + 1, 0, _H - 1).astype(np.int64)
    y0 = np.clip(y0f, 0, _W - 1).astype(np.int64)
    y1 = np.clip(y0f + 1, 0, _W - 1).astype(np.int64)
    i = np.stack([(x0 * _W + y0).ravel(), (x0 * _W + y1).ravel(),
                  (x1 * _W + y0).ravel(), (x1 * _W + y1).ravel()])
    w = np.stack([((1 - fx) * (1 - fy)).ravel(), ((1 - fx) * fy).ravel(),
                  (fx * (1 - fy)).ravel(), (fx * fy).ravel()])
    boff = (np.arange(_B) * (_H * _W))[None, :, None]
    idx = (i[:, None, :] + boff).reshape(4 * _NROWS).astype(np.int32)
    wts = np.zeros((_NROWS, 8), dtype=np.float32)
    wts[:, :4] = np.broadcast_to(w.T[None], (_B, _H * _W, 4)).reshape(_NROWS, 4)
    _tables_cache = (idx, wts.reshape(-1))
    return _tables_cache


_CP = 128


def _sc_body(rows_hbm, idx_hbm, wts_hbm, out_hbm,
             idx_v0, idx_v1, w_v0, w_v1, tap_v0, tap_v1, out_v0, out_v1,
             sem0, sem1):
    wid = lax.axis_index("s") * 2 + lax.axis_index("c")
    bufs = ((idx_v0, w_v0, tap_v0, out_v0, sem0),
            (idx_v1, w_v1, tap_v1, out_v1, sem1))

    def start(j, b):
        idx_v, w_v, tap_v, _, sem = bufs[b]
        base = wid * _PER_W + j * _CHUNK
        for t in range(4):
            pltpu.sync_copy(idx_hbm.at[pl.ds(t * _NROWS + base, _CHUNK)],
                            idx_v.at[pl.ds(t * _CHUNK, _CHUNK)])
        pltpu.sync_copy(wts_hbm.at[pl.ds(base * 8, _CHUNK * 8)],
                        w_v.at[pl.ds(0, _CHUNK * 8)])
        for t in range(4):
            pltpu.async_copy(rows_hbm.at[idx_v.at[pl.ds(t * _CHUNK, _CHUNK)]],
                             tap_v.at[t], sem)

    def finish(j, b):
        idx_v, w_v, tap_v, out_v, sem = bufs[b]
        base = wid * _PER_W + j * _CHUNK
        for t in range(4):
            pltpu.make_async_copy(
                rows_hbm.at[idx_v.at[pl.ds(t * _CHUNK, _CHUNK)]],
                tap_v.at[t], sem).wait()

        @plsc.parallel_loop(0, _CHUNK, unroll=4)
        def row(k):
            wvec = w_v[pl.ds(k * 8, 16)]
            w0 = wvec[0]; w1 = wvec[1]; w2 = wvec[2]; w3 = wvec[3]
            for c in range(_C // 16):
                s = pl.ds(c * 16, 16)
                out_v[k, s] = (tap_v[0, k, s] * w0 + tap_v[1, k, s] * w1
                               + tap_v[2, k, s] * w2 + tap_v[3, k, s] * w3)
        pltpu.sync_copy(out_v, out_hbm.at[pl.ds(base, _CHUNK)])

    start(0, 0)

    def pair(j2, carry):
        e = j2 * 2
        start(e + 1, 1)
        finish(e, 0)

        @pl.when(j2 < _NCHUNK // 2 - 1)
        def _():
            start(e + 2, 0)

        finish(e + 1, 1)
        return carry

    lax.fori_loop(0, _NCHUNK // 2, pair, 0)


@jax.jit
def _polar_sc(rows, idx, wts):
    mesh = plsc.VectorSubcoreMesh(core_axis_name="c", subcore_axis_name="s")
    f = functools.partial(
        pl.kernel,
        mesh=mesh,
        out_type=jax.ShapeDtypeStruct((_NROWS, _CP), jnp.float32),
        scratch_types=[
            pltpu.VMEM((4 * _CHUNK,), jnp.int32),
            pltpu.VMEM((4 * _CHUNK,), jnp.int32),
            pltpu.VMEM((_CHUNK * 8 + 16,), jnp.float32),
            pltpu.VMEM((_CHUNK * 8 + 16,), jnp.float32),
            pltpu.VMEM((4, _CHUNK, _CP), jnp.float32),
            pltpu.VMEM((4, _CHUNK, _CP), jnp.float32),
            pltpu.VMEM((_CHUNK, _CP), jnp.float32),
            pltpu.VMEM((_CHUNK, _CP), jnp.float32),
            pltpu.SemaphoreType.DMA,
            pltpu.SemaphoreType.DMA,
        ],
        compiler_params=pltpu.CompilerParams(use_tc_tiling_on_sc=False),
    )(_sc_body)
    return f(rows, idx, wts)


def kernel(inputs):
    assert inputs.shape == (_B, _H, _W, _C)
    rows = jnp.pad(inputs, ((0, 0), (0, 0), (0, 0), (0, _CP - _C))).reshape(_NROWS, _CP)
    idx_np, wts_np = _build_tables()
    out = _polar_sc(rows, jnp.asarray(idx_np), jnp.asarray(wts_np))
    return out[:, :_C].reshape(_B, _H, _W, _C)


# speedup vs baseline: 3.5964x; 1.0318x over previous
import functools

import numpy as np
import jax
import jax.numpy as jnp
from jax import lax
from jax.experimental import pallas as pl
from jax.experimental.pallas import tpu as pltpu
from jax.experimental.pallas import tpu_sc as plsc

_B, _H, _W, _C = 4, 224, 224, 96
_OUT_SHAPE = (224, 224)
_CENTER = (112.0, 112.0)
_MAX_RADIUS = 112.0

_NW = 32
_NROWS = _B * _H * _W
_PER_W = _NROWS // _NW
_CHUNK = 64
_NCHUNK = _PER_W // _CHUNK

_tables_cache = None


def _build_tables():
    global _tables_cache
    if _tables_cache is not None:
        return _tables_cache
    radius = np.linspace(0.0, _MAX_RADIUS, _OUT_SHAPE[0], dtype=np.float32).astype(np.float64)
    theta = np.linspace(0.0, 2.0 * np.pi, _OUT_SHAPE[1], endpoint=False,
                        dtype=np.float32).astype(np.float64)
    c1 = _CENTER[0] + radius[:, None] * np.cos(theta)[None, :]
    c2 = _CENTER[1] + radius[:, None] * np.sin(theta)[None, :]
    x0f = np.floor(c1); fx = c1 - x0f
    y0f = np.floor(c2); fy = c2 - y0f
    x0 = np.clip(x0f, 0, _H - 1).astype(np.int64)
    x1 = np.clip(x0f + 1, 0, _H - 1).astype(np.int64)
    y0 = np.clip(y0f, 0, _W - 1).astype(np.int64)
    y1 = np.clip(y0f + 1, 0, _W - 1).astype(np.int64)
    i = np.stack([(x0 * _W + y0).ravel(), (x0 * _W + y1).ravel(),
                  (x1 * _W + y0).ravel(), (x1 * _W + y1).ravel()])
    w = np.stack([((1 - fx) * (1 - fy)).ravel(), ((1 - fx) * fy).ravel(),
                  (fx * (1 - fy)).ravel(), (fx * fy).ravel()])
    boff = (np.arange(_B) * (_H * _W))[None, :, None]
    idx = (i[:, None, :] + boff).reshape(4 * _NROWS).astype(np.int32)
    wts = np.ascontiguousarray(
        np.broadcast_to(w.T[None], (_B, _H * _W, 4)).reshape(_NROWS, 4)
    ).astype(np.float32)
    _tables_cache = (idx, wts.reshape(-1))
    return _tables_cache


_CP = 128


def _sc_body(rows_hbm, idx_hbm, wts_hbm, out_hbm,
             idx_v0, idx_v1, w_v0, w_v1, tap_v0, tap_v1, out_v0, out_v1,
             sem0, sem1):
    wid = lax.axis_index("s") * 2 + lax.axis_index("c")
    bufs = ((idx_v0, w_v0, tap_v0, out_v0, sem0),
            (idx_v1, w_v1, tap_v1, out_v1, sem1))

    def start(j, b):
        idx_v, w_v, tap_v, _, sem = bufs[b]
        base = wid * _PER_W + j * _CHUNK
        for t in range(4):
            pltpu.sync_copy(idx_hbm.at[pl.ds(t * _NROWS + base, _CHUNK)],
                            idx_v.at[pl.ds(t * _CHUNK, _CHUNK)])
        pltpu.sync_copy(wts_hbm.at[pl.ds(base * 4, _CHUNK * 4)],
                        w_v.at[pl.ds(0, _CHUNK * 4)])
        for t in range(4):
            pltpu.async_copy(rows_hbm.at[idx_v.at[pl.ds(t * _CHUNK, _CHUNK)]],
                             tap_v.at[t], sem)

    def finish(j, b):
        idx_v, w_v, tap_v, out_v, sem = bufs[b]
        base = wid * _PER_W + j * _CHUNK
        for t in range(4):
            pltpu.make_async_copy(
                rows_hbm.at[idx_v.at[pl.ds(t * _CHUNK, _CHUNK)]],
                tap_v.at[t], sem).wait()

        @plsc.parallel_loop(0, _CHUNK, unroll=4)
        def row(k):
            wvec = w_v[pl.ds(k * 4, 16)]
            w0 = wvec[0]; w1 = wvec[1]; w2 = wvec[2]; w3 = wvec[3]
            for c in range(_C // 16):
                s = pl.ds(c * 16, 16)
                out_v[k, s] = (tap_v[0, k, s] * w0 + tap_v[1, k, s] * w1
                               + tap_v[2, k, s] * w2 + tap_v[3, k, s] * w3)
        pltpu.sync_copy(out_v, out_hbm.at[pl.ds(base, _CHUNK)])

    start(0, 0)

    def pair(j2, carry):
        e = j2 * 2
        start(e + 1, 1)
        finish(e, 0)

        @pl.when(j2 < _NCHUNK // 2 - 1)
        def _():
            start(e + 2, 0)

        finish(e + 1, 1)
        return carry

    lax.fori_loop(0, _NCHUNK // 2, pair, 0)


@jax.jit
def _polar_sc(rows, idx, wts):
    mesh = plsc.VectorSubcoreMesh(core_axis_name="c", subcore_axis_name="s")
    f = functools.partial(
        pl.kernel,
        mesh=mesh,
        out_type=jax.ShapeDtypeStruct((_NROWS, _CP), jnp.float32),
        scratch_types=[
            pltpu.VMEM((4 * _CHUNK,), jnp.int32),
            pltpu.VMEM((4 * _CHUNK,), jnp.int32),
            pltpu.VMEM((_CHUNK * 4 + 16,), jnp.float32),
            pltpu.VMEM((_CHUNK * 4 + 16,), jnp.float32),
            pltpu.VMEM((4, _CHUNK, _CP), jnp.float32),
            pltpu.VMEM((4, _CHUNK, _CP), jnp.float32),
            pltpu.VMEM((_CHUNK, _CP), jnp.float32),
            pltpu.VMEM((_CHUNK, _CP), jnp.float32),
            pltpu.SemaphoreType.DMA,
            pltpu.SemaphoreType.DMA,
        ],
        compiler_params=pltpu.CompilerParams(use_tc_tiling_on_sc=False),
    )(_sc_body)
    return f(rows, idx, wts)


def kernel(inputs):
    assert inputs.shape == (_B, _H, _W, _C)
    rows = jnp.pad(inputs, ((0, 0), (0, 0), (0, 0), (0, _CP - _C))).reshape(_NROWS, _CP)
    idx_np, wts_np = _build_tables()
    out = _polar_sc(rows, jnp.asarray(idx_np), jnp.asarray(wts_np))
    return out[:, :_C].reshape(_B, _H, _W, _C)
